# baseline (speedup 1.0000x reference)
import jax
import jax.numpy as jnp
from jax import lax
from jax.experimental import pallas as pl
from jax.experimental.pallas import tpu as pltpu
from jax.experimental.pallas import tpu_sc as plsc

BATCH = 4096
MAX_SEQ = 200
EMBED = 64
VOCAB = 1000000
LANES = 16

_info = plsc.get_sparse_core_info()
NUM_CORES = _info.num_cores
NUM_SUBCORES = _info.num_subcores
NUM_WORKERS = NUM_CORES * NUM_SUBCORES

TOTAL_ROWS = BATCH * MAX_SEQ
ROWS_PER_WORKER = TOTAL_ROWS // NUM_WORKERS
CHUNK = 400
NCHUNKS = ROWS_PER_WORKER // CHUNK
VECS_PER_ROW = EMBED // LANES


NBLK = (VOCAB + 127) // 128
FULL_BLOCKS = VOCAB // 128
BLK_PER_WORKER = (NBLK + NUM_WORKERS - 1) // NUM_WORKERS


def _transpose_body(tbl_t_hbm, out_hbm, in0, in1, tr0, tr1, is0, is1, os0, os1):
    wid = lax.axis_index("s") * NUM_CORES + lax.axis_index("c")
    bufs = ((in0, tr0, is0, os0), (in1, tr1, is1, os1))
    lane128 = lax.iota(jnp.int32, LANES) * 128

    def start_in(bi, b):
        in_v, _, isem, _ = bufs[b]
        blk = wid + NUM_WORKERS * bi
        pltpu.async_copy(tbl_t_hbm.at[:, pl.ds(blk * 128, 128)], in_v, isem)

    def wait_in(bi, b):
        in_v, _, isem, _ = bufs[b]
        pltpu.make_async_copy(
            tbl_t_hbm.at[:, pl.ds(0, 128)], in_v, isem).wait()

    def start_out(bi, b):
        _, tr_v, _, osem = bufs[b]
        blk = wid + NUM_WORKERS * bi

        @pl.when(blk < FULL_BLOCKS)
        def _():
            pltpu.async_copy(tr_v, out_hbm.at[pl.ds(blk * 64, 64)], osem)

        @pl.when(blk == FULL_BLOCKS)
        def _():
            pltpu.async_copy(tr_v.at[pl.ds(0, 32), :],
                             out_hbm.at[pl.ds(FULL_BLOCKS * 64, 32)], osem)

    def wait_out(bi, b):
        _, tr_v, _, osem = bufs[b]
        blk = wid + NUM_WORKERS * bi

        @pl.when(blk < FULL_BLOCKS)
        def _():
            pltpu.make_async_copy(
                tr_v, out_hbm.at[pl.ds(0, 64)], osem).wait()

        @pl.when(blk == FULL_BLOCKS)
        def _():
            pltpu.make_async_copy(
                tr_v.at[pl.ds(0, 32), :], out_hbm.at[pl.ds(0, 32)], osem).wait()

    def transpose(bi, b):
        in_v, tr_v, _, _ = bufs[b]

        @plsc.parallel_loop(0, 128, 1, unroll=4)
        def _row(r):
            pr = r // 2
            col0 = (r % 2) * EMBED
            rvec = jnp.full((LANES,), r, jnp.int32)
            for j in range(VECS_PER_ROW):
                cvec = lax.iota(jnp.int32, LANES) + (j * LANES)
                v = plsc.load_gather(in_v, [cvec, rvec])
                tr_v[pr, pl.ds(col0 + j * LANES, LANES)] = v

    start_in(0, 0)
    start_in(1, 1)

    def blk_body(i, carry):
        for b in (0, 1):
            bi = 2 * i + b
            blk = wid + NUM_WORKERS * bi

            @pl.when(blk <= FULL_BLOCKS)
            def _():
                wait_in(bi, b)

                @pl.when(bi >= 2)
                def _():
                    wait_out(bi - 2, b)

                transpose(bi, b)
                start_out(bi, b)

            @pl.when(wid + NUM_WORKERS * (bi + 2) <= FULL_BLOCKS)
            def _():
                start_in(bi + 2, b)
        return carry

    nb_pairs = (BLK_PER_WORKER + 1) // 2
    lax.fori_loop(0, nb_pairs, blk_body, 0)
    for b in (0, 1):
        bi_last = 2 * nb_pairs - 2 + b

        @pl.when(wid + NUM_WORKERS * bi_last <= FULL_BLOCKS)
        def _():
            wait_out(bi_last, b)


@jax.jit
def _transpose_table(tbl_t):
    mesh = plsc.VectorSubcoreMesh(core_axis_name="c", subcore_axis_name="s")
    run = pl.kernel(
        _transpose_body,
        out_type=jax.ShapeDtypeStruct((VOCAB // 2, 2 * EMBED), jnp.float32),
        mesh=mesh,
        scratch_types=[
            pltpu.VMEM((EMBED, 128), jnp.float32),
            pltpu.VMEM((EMBED, 128), jnp.float32),
            pltpu.VMEM((EMBED, 128), jnp.float32),
            pltpu.VMEM((EMBED, 128), jnp.float32),
            pltpu.SemaphoreType.DMA,
            pltpu.SemaphoreType.DMA,
            pltpu.SemaphoreType.DMA,
            pltpu.SemaphoreType.DMA,
        ],
        compiler_params=pltpu.CompilerParams(
            use_tc_tiling_on_sc=True, needs_layout_passes=False),
    )
    return run(tbl_t)


def _sc_body(table_hbm, idx_hbm, pos_hbm, out_hbm,
             idx0, idx1, rows0, rows1, pos_v,
             isem0, isem1, gsem0, gsem1, osem0, osem1):
    wid = lax.axis_index("s") * NUM_CORES + lax.axis_index("c")
    base = wid * ROWS_PER_WORKER

    bufs = ((idx0, rows0, isem0, gsem0, osem0),
            (idx1, rows1, isem1, gsem1, osem1))

    def start_idx(g, b):
        idx_v, _, isem, _, _ = bufs[b]
        pltpu.async_copy(idx_hbm.at[pl.ds(base + g * CHUNK, CHUNK)], idx_v, isem)

    def wait_idx(b):
        idx_v, _, isem, _, _ = bufs[b]
        pltpu.make_async_copy(idx_hbm.at[pl.ds(base, CHUNK)], idx_v, isem).wait()

    def start_gather(b):
        idx_v, rows_v, _, gsem, _ = bufs[b]
        pltpu.async_copy(table_hbm.at[idx_v], rows_v, gsem)

    def wait_gather(b):
        idx_v, rows_v, _, gsem, _ = bufs[b]
        pltpu.make_async_copy(table_hbm.at[idx_v], rows_v, gsem).wait()

    def start_scatter(g, b):
        _, rows_v, _, _, osem = bufs[b]
        pltpu.async_copy(rows_v, out_hbm.at[pl.ds(base + g * CHUNK, CHUNK)], osem)

    def wait_scatter(b):
        _, rows_v, _, _, osem = bufs[b]
        pltpu.make_async_copy(rows_v, out_hbm.at[pl.ds(base, CHUNK)], osem).wait()

    def add_pos(b):
        _, rows_v, _, _, _ = bufs[b]

        @plsc.parallel_loop(0, CHUNK, 1, unroll=8)
        def _body(r):
            for j in range(VECS_PER_ROW):
                sl = pl.ds(j * LANES, LANES)
                plsc.addupdate(rows_v.at[r, sl], pos_v[r, sl])

    pltpu.sync_copy(pos_hbm, pos_v)
    start_idx(0, 0)
    start_idx(1, 1)
    wait_idx(0)
    start_gather(0)

    def pair_body(i, carry):
        for b in (0, 1):
            g = 2 * i + b
            wait_gather(b)
            @pl.when(g + 2 < NCHUNKS)
            def _():
                start_idx(g + 2, b)
            @pl.when(g >= 1)
            def _():
                wait_scatter(1 - b)
            @pl.when(g + 1 < NCHUNKS)
            def _():
                wait_idx(1 - b)
                start_gather(1 - b)
            add_pos(b)
            start_scatter(g, b)
        return carry

    lax.fori_loop(0, NCHUNKS // 2, pair_body, 0)
    wait_scatter((NCHUNKS - 1) % 2)


@jax.jit
def _embed(idx_flat, token_table, pos_tiled):
    mesh = plsc.VectorSubcoreMesh(core_axis_name="c", subcore_axis_name="s")
    run = pl.kernel(
        _sc_body,
        out_type=jax.ShapeDtypeStruct((TOTAL_ROWS, EMBED), jnp.float32),
        mesh=mesh,
        scratch_types=[
            pltpu.VMEM((CHUNK,), jnp.int32),
            pltpu.VMEM((CHUNK,), jnp.int32),
            pltpu.VMEM((CHUNK, EMBED), jnp.float32),
            pltpu.VMEM((CHUNK, EMBED), jnp.float32),
            pltpu.VMEM((CHUNK, EMBED), jnp.float32),
            pltpu.SemaphoreType.DMA,
            pltpu.SemaphoreType.DMA,
            pltpu.SemaphoreType.DMA,
            pltpu.SemaphoreType.DMA,
            pltpu.SemaphoreType.DMA,
            pltpu.SemaphoreType.DMA,
        ],
        compiler_params=pltpu.CompilerParams(use_tc_tiling_on_sc=False),
    )
    return run(token_table, idx_flat, pos_tiled)


def kernel(inputs, token_table, position_table):
    idx_flat = inputs.reshape(-1).astype(jnp.int32)
    pos_tiled = jnp.tile(position_table, (CHUNK // MAX_SEQ, 1))
    tbl_packed = _transpose_table(token_table.T)
    tbl_lin = jnp.reshape(tbl_packed, (VOCAB, EMBED))
    out = _embed(idx_flat, tbl_lin, pos_tiled)
    return out.reshape(BATCH, MAX_SEQ, EMBED)
